# Initial kernel scaffold; baseline (speedup 1.0000x reference)
import jax
import jax.numpy as jnp
from jax.experimental import pallas as pl


def kernel(x, W1, W2, coefficients):
    raise NotImplementedError("write your pallas kernel here")


# speedup vs baseline: 59.6424x; 59.6424x over previous
import functools

import jax
import jax.numpy as jnp
from jax.experimental import pallas as pl
from jax.experimental.pallas import tpu as pltpu

_NK = 21
_XMIN = -1.0
_XMAX = 1.0
_STEP = (_XMAX - _XMIN) / (_NK - 1)
_KS = 5


def _row_chunk(H):
    for c in (48, 64, 96):
        if H % c == 0:
            return c
    return H


def _accumulate_taps(x_ref, w_ref, o_ref):
    oc_t = o_ref.shape[1]
    ic_t = x_ref.shape[1]
    H = o_ref.shape[2]
    W = o_ref.shape[3]
    CH = _row_chunk(H)
    ic = pl.program_id(2)
    for h0 in range(0, H, CH):
        acc = jnp.zeros((oc_t, CH * W), jnp.float32)
        for dy in range(_KS):
            for dx in range(_KS):
                xs = x_ref[0, :, h0 + dy:h0 + dy + CH, dx:dx + W]
                wt = w_ref[:, :, dy, dx]
                acc = acc + jax.lax.dot_general(
                    wt, xs.reshape(ic_t, CH * W), (((1,), (0,)), ((), ())),
                    preferred_element_type=jnp.float32)
        acc = acc.reshape(oc_t, CH, W)

        @pl.when(ic == 0)
        def _(acc=acc, h0=h0):
            o_ref[0, :, h0:h0 + CH, :] = acc

        @pl.when(ic > 0)
        def _(acc=acc, h0=h0):
            o_ref[0, :, h0:h0 + CH, :] = o_ref[0, :, h0:h0 + CH, :] + acc


def _conv_kernel(x_ref, w_ref, o_ref):
    _accumulate_taps(x_ref, w_ref, o_ref)


def _conv_spline_kernel(x_ref, w_ref, c0_ref, m_ref, o_ref, *, nic):
    _accumulate_taps(x_ref, w_ref, o_ref)
    ic = pl.program_id(2)

    @pl.when(ic == nic - 1)
    def _():
        oc_t = o_ref.shape[1]
        H = o_ref.shape[2]
        CH = _row_chunk(H)
        c0 = c0_ref[...].reshape(oc_t, 1, 1)
        for h0 in range(0, H, CH):
            u = o_ref[0, :, h0:h0 + CH, :]
            s = jnp.broadcast_to(c0, u.shape)
            for k in range(_NK - 1):
                tk = _XMIN + k * _STEP
                mk = m_ref[:, k:k + 1].reshape(oc_t, 1, 1)
                s = s + mk * (jnp.clip(u, tk, tk + _STEP) - tk)
            o_ref[0, :, h0:h0 + CH, :] = s


def _run_conv(xp, w, oc_t, ic_t, spline=None):
    B, IC, Hp, Wp = xp.shape
    OC = w.shape[0]
    H, W = Hp - (_KS - 1), Wp - (_KS - 1)
    nic = IC // ic_t
    grid = (B, OC // oc_t, nic)

    x_spec = pl.BlockSpec((1, ic_t, Hp, Wp), lambda b, o, i: (b, i, 0, 0))
    w_spec = pl.BlockSpec((oc_t, ic_t, _KS, _KS), lambda b, o, i: (o, i, 0, 0))
    o_spec = pl.BlockSpec((1, oc_t, H, W), lambda b, o, i: (b, o, 0, 0))

    if spline is None:
        kern = _conv_kernel
        in_specs = [x_spec, w_spec]
        args = (xp, w)
    else:
        c0, m = spline
        kern = functools.partial(_conv_spline_kernel, nic=nic)
        in_specs = [
            x_spec, w_spec,
            pl.BlockSpec((oc_t, 1), lambda b, o, i: (o, 0)),
            pl.BlockSpec((oc_t, _NK - 1), lambda b, o, i: (o, 0)),
        ]
        args = (xp, w, c0, m)

    return pl.pallas_call(
        kern,
        grid=grid,
        in_specs=in_specs,
        out_specs=o_spec,
        out_shape=jax.ShapeDtypeStruct((B, OC, H, W), jnp.float32),
        compiler_params=pltpu.CompilerParams(
            vmem_limit_bytes=60 * 1024 * 1024),
    )(*args)


def _pad2(t):
    p = _KS // 2
    return jnp.pad(t, ((0, 0), (0, 0), (p, p), (p, p)))


def kernel(x, W1, W2, coefficients):
    W1z = W1 - jnp.mean(W1, axis=(1, 2, 3), keepdims=True)
    W2z = W2 - jnp.mean(W2, axis=(1, 2, 3), keepdims=True)

    c = coefficients
    slopes = (c[:, 1:] - c[:, :-1]) / _STEP
    slopes = slopes.at[:, 0].set(0.0).at[:, -1].set(0.0)
    new = jnp.concatenate(
        [jnp.zeros((c.shape[0], 1), c.dtype),
         jnp.cumsum(slopes, axis=1) * _STEP], axis=1)
    pc = new + jnp.mean(c - new, axis=1, keepdims=True)
    m = (pc[:, 1:] - pc[:, :-1]) / _STEP
    c0 = pc[:, :1]

    W2T = jnp.transpose(W2z, (1, 0, 2, 3))[:, :, ::-1, ::-1]
    W1T = jnp.transpose(W1z, (1, 0, 2, 3))[:, :, ::-1, ::-1]

    u1 = _run_conv(_pad2(x), W1z, oc_t=16, ic_t=1)
    s = _run_conv(_pad2(u1), W2z, oc_t=16, ic_t=8, spline=(c0, m))
    v = _run_conv(_pad2(s), W2T, oc_t=16, ic_t=8)
    out = _run_conv(_pad2(v), W1T, oc_t=1, ic_t=8)
    return out
